# speedup vs baseline: 2.5535x; 2.5535x over previous
import jax
import jax.numpy as jnp
from jax.experimental import pallas as pl

H = 512
W = 512


def _stencil_kernel(tab_ref, x_ref, o_ref):
    x = x_ref[0, 0]
    a0 = tab_ref[0, 0, 0]
    a1 = tab_ref[0, 0, 1]
    a2 = tab_ref[0, 0, 2]
    a3 = tab_ref[0, 0, 3]
    a4 = tab_ref[0, 0, 4]
    a5 = tab_ref[0, 0, 5]
    b = tab_ref[0, 0, 6]

    zr1 = jnp.zeros((1, W), jnp.float32)
    zr2 = jnp.zeros((2, W), jnp.float32)
    u1 = jnp.concatenate([x[1:], zr1], 0) + jnp.concatenate([zr1, x[:-1]], 0)
    u2 = jnp.concatenate([x[2:], zr2], 0) + jnp.concatenate([zr2, x[:-2]], 0)

    zc1 = jnp.zeros((H, 1), jnp.float32)
    zc2 = jnp.zeros((H, 2), jnp.float32)

    def A1(u):
        return (jnp.concatenate([u[:, 1:], zc1], 1)
                + jnp.concatenate([zc1, u[:, :-1]], 1))

    def A2(u):
        return (jnp.concatenate([u[:, 2:], zc2], 1)
                + jnp.concatenate([zc2, u[:, :-2]], 1))

    t1 = A1(x) + u1
    t2 = A1(u1)
    t3 = A2(x) + u2
    t4 = A1(u2) + A2(u1)
    t5 = A2(u2)

    o_ref[0, 0] = (a0 * x + a1 * t1 + a2 * t2 + a3 * t3 + a4 * t4
                   + a5 * t5 + b)


def kernel(x, weight, bias, psi_loc):
    n, c, h, w = x.shape
    full7 = jnp.einsum('kxy,ok->oxy', psi_loc, weight[:, 0, :])
    tab = jnp.stack([
        full7[:, 3, 3],
        full7[:, 3, 4],
        full7[:, 2, 4],
        full7[:, 3, 5],
        full7[:, 2, 5],
        full7[:, 1, 5],
        bias,
        jnp.zeros_like(bias),
    ], axis=-1)
    tab = tab.reshape(c, 1, 8)

    out = pl.pallas_call(
        _stencil_kernel,
        grid=(n, c),
        in_specs=[
            pl.BlockSpec((1, 1, 8), lambda i, j: (j, 0, 0)),
            pl.BlockSpec((1, 1, h, w), lambda i, j: (i, j, 0, 0)),
        ],
        out_specs=pl.BlockSpec((1, 1, h, w), lambda i, j: (i, j, 0, 0)),
        out_shape=jax.ShapeDtypeStruct((n, c, h, w), jnp.float32),
    )(tab, x)
    return out
